# baseline (speedup 1.0000x reference)
import functools

import jax
import jax.numpy as jnp
from jax import lax
from jax.experimental import pallas as pl
from jax.experimental.pallas import tpu as pltpu
from jax.experimental.pallas import tpu_sc as plsc

_info = plsc.get_sparse_core_info()
_NC, _NS = _info.num_cores, _info.num_subcores
_NW = _NC * _NS


def _make_copy_kernel(t: int, d: int):
    assert t % _NW == 0
    rows_w = t // _NW

    mesh = plsc.VectorSubcoreMesh(core_axis_name="c", subcore_axis_name="s")

    chunk = 16
    assert rows_w % chunk == 0
    nchunks = rows_w // chunk

    @functools.partial(
        pl.kernel,
        mesh=mesh,
        out_type=jax.ShapeDtypeStruct((1, t, d), jnp.float32),
        scratch_types=[
            pltpu.VMEM((nchunks, chunk, d), jnp.float32),
            pltpu.SemaphoreType.DMA,
            pltpu.SemaphoreType.DMA,
        ],
    )
    def copy_rows(emb_hbm, out_hbm, buf, gsem, ssem):
        wid = lax.axis_index("s") * _NC + lax.axis_index("c")
        base = wid * rows_w
        gathers = [
            pltpu.async_copy(
                emb_hbm.at[pl.ds(base + i * chunk, chunk)], buf.at[i], gsem
            )
            for i in range(nchunks)
        ]
        scatters = []
        for i in range(nchunks):
            gathers[i].wait()
            scatters.append(
                pltpu.async_copy(
                    buf.at[i], out_hbm.at[0, pl.ds(base + i * chunk, chunk)], ssem
                )
            )
        for s in scatters:
            s.wait()

    return copy_rows


def kernel(input_ids, emb_weight):
    b, t = input_ids.shape
    d = emb_weight.shape[1]
    return _make_copy_kernel(t, d)(emb_weight)


# speedup vs baseline: 1.0223x; 1.0223x over previous
import functools

import jax
import jax.numpy as jnp
from jax import lax
from jax.experimental import pallas as pl
from jax.experimental.pallas import tpu as pltpu
from jax.experimental.pallas import tpu_sc as plsc

_info = plsc.get_sparse_core_info()
_NC, _NS = _info.num_cores, _info.num_subcores
_NW = _NC * _NS


def _make_copy_kernel(t: int, d: int):
    assert t % _NW == 0
    rows_w = t // _NW

    mesh = plsc.VectorSubcoreMesh(core_axis_name="c", subcore_axis_name="s")

    chunk = rows_w
    assert rows_w % chunk == 0
    nchunks = rows_w // chunk

    @functools.partial(
        pl.kernel,
        mesh=mesh,
        out_type=jax.ShapeDtypeStruct((1, t, d), jnp.float32),
        scratch_types=[
            pltpu.VMEM((nchunks, chunk, d), jnp.float32),
            pltpu.SemaphoreType.DMA,
            pltpu.SemaphoreType.DMA,
        ],
    )
    def copy_rows(emb_hbm, out_hbm, buf, gsem, ssem):
        wid = lax.axis_index("s") * _NC + lax.axis_index("c")
        base = wid * rows_w
        gathers = [
            pltpu.async_copy(
                emb_hbm.at[pl.ds(base + i * chunk, chunk)], buf.at[i], gsem
            )
            for i in range(nchunks)
        ]
        scatters = []
        for i in range(nchunks):
            gathers[i].wait()
            scatters.append(
                pltpu.async_copy(
                    buf.at[i], out_hbm.at[0, pl.ds(base + i * chunk, chunk)], ssem
                )
            )
        for s in scatters:
            s.wait()

    return copy_rows


def kernel(input_ids, emb_weight):
    b, t = input_ids.shape
    d = emb_weight.shape[1]
    return _make_copy_kernel(t, d)(emb_weight)
